# baseline (speedup 1.0000x reference)
import functools

import jax
import jax.numpy as jnp
from jax import lax
from jax.experimental import pallas as pl
from jax.experimental.pallas import tpu as pltpu
from jax.experimental.pallas import tpu_sc as plsc

CHUNK = 128
BLKP = 16384


def _relayout_body(t_ref, eye_ref, out_ref):
    x = t_ref[...]
    y = jnp.concatenate([x[:, :BLKP], x[:, BLKP:]], axis=0)
    out_ref[...] = lax.dot_general(
        y, eye_ref[...], (((0,), (0,)), ((), ())),
        preferred_element_type=jnp.float32)


def _relayout(tt):
    E, N = tt.shape
    W = 2 * BLKP
    grid = (N + W - 1) // W
    return pl.pallas_call(
        _relayout_body,
        grid=(grid,),
        in_specs=[
            pl.BlockSpec((E, W), lambda i: (0, i)),
            pl.BlockSpec((2 * E, 2 * E), lambda i: (0, 0)),
        ],
        out_specs=pl.BlockSpec((BLKP, 2 * E), lambda i: (i, 0)),
        out_shape=jax.ShapeDtypeStruct((grid * BLKP, 2 * E), jnp.float32),
        compiler_params=pltpu.CompilerParams(
            dimension_semantics=("parallel",)),
    )(tt, jnp.eye(2 * E, dtype=jnp.float32))


def _map_idx(idx):
    W = 2 * BLKP
    q = idx // W
    s = idx % W
    p = q * BLKP + jnp.where(s < BLKP, s, s - BLKP)
    half = (s >= BLKP).astype(jnp.int32)
    return p, half


@functools.partial(jax.jit, static_argnums=(2,))
def _sc_gather(t2, idx, B):
    info = plsc.get_sparse_core_info()
    NW = info.num_cores * info.num_subcores
    b_per_w = B // NW
    n_ch = b_per_w // CHUNK
    mesh = plsc.VectorSubcoreMesh(core_axis_name="c", subcore_axis_name="s")

    @functools.partial(
        pl.kernel,
        mesh=mesh,
        out_type=jax.ShapeDtypeStruct((B, 128), jnp.float32),
        scratch_types=[
            pltpu.VMEM((n_ch, CHUNK), jnp.int32),
            pltpu.VMEM((2, CHUNK, 128), jnp.float32),
            pltpu.SemaphoreType.DMA,
            pltpu.SemaphoreType.DMA,
        ],
    )
    def k(t_hbm, ix_hbm, g_hbm, idx_v, buf, gsem, wsem):
        wid = lax.axis_index("s") * info.num_cores + lax.axis_index("c")
        base = wid * b_per_w
        pltpu.sync_copy(ix_hbm.at[wid], idx_v)
        writes = [None, None]
        for j in range(n_ch):
            b = j % 2
            if writes[b] is not None:
                writes[b].wait()
            g = pltpu.async_copy(t_hbm.at[idx_v.at[j]], buf.at[b], gsem)
            g.wait()
            dst = pl.ds(base + j * CHUNK, CHUNK)
            writes[b] = pltpu.async_copy(buf.at[b], g_hbm.at[dst], wsem)
        for w in writes:
            if w is not None:
                w.wait()

    ix3 = idx.reshape(NW, n_ch, CHUNK)
    return k(t2, ix3)


def _mlp_body(gu_ref, gm_ref, up_ref, mp_ref, w1u_ref, w1m_ref, b1_ref,
              w2_ref, b2_ref, out_ref):
    ue = jnp.where(up_ref[...] == 1, gu_ref[:, 64:], gu_ref[:, :64])
    me = jnp.where(mp_ref[...] == 1, gm_ref[:, 64:], gm_ref[:, :64])
    h = jnp.dot(ue, w1u_ref[...], preferred_element_type=jnp.float32)
    h = h + jnp.dot(me, w1m_ref[...], preferred_element_type=jnp.float32)
    h = jnp.maximum(h + b1_ref[...], 0.0)
    out_ref[...] = jnp.dot(h, w2_ref[...], preferred_element_type=jnp.float32) + b2_ref[...]


def _tc_mlp(gu, gm, up, mp, w1u, w1m, b1, w2, b2):
    B = gu.shape[0]
    H = w1u.shape[1]
    BLK = 2048
    return pl.pallas_call(
        _mlp_body,
        grid=(B // BLK,),
        in_specs=[
            pl.BlockSpec((BLK, 128), lambda i: (i, 0)),
            pl.BlockSpec((BLK, 128), lambda i: (i, 0)),
            pl.BlockSpec((BLK, 1), lambda i: (i, 0)),
            pl.BlockSpec((BLK, 1), lambda i: (i, 0)),
            pl.BlockSpec((64, H), lambda i: (0, 0)),
            pl.BlockSpec((64, H), lambda i: (0, 0)),
            pl.BlockSpec((1, H), lambda i: (0, 0)),
            pl.BlockSpec((H, 1), lambda i: (0, 0)),
            pl.BlockSpec((1, 1), lambda i: (0, 0)),
        ],
        out_specs=pl.BlockSpec((BLK, 1), lambda i: (i, 0)),
        out_shape=jax.ShapeDtypeStruct((B, 1), jnp.float32),
        compiler_params=pltpu.CompilerParams(
            dimension_semantics=("parallel",)),
    )(gu, gm, up, mp, w1u, w1m, b1, w2, b2)


def kernel(users, movies, user_table, movie_table, W1, b1, W2, b2):
    B = users.shape[0]
    D = user_table.shape[1]
    users = users.astype(jnp.int32)
    movies = movies.astype(jnp.int32)
    pu, up = _map_idx(users)
    pm, mp = _map_idx(movies)
    rm_u = _relayout(user_table.T)
    gu = _sc_gather(rm_u, pu, B)
    rm_m = _relayout(movie_table.T)
    gm = _sc_gather(rm_m, pm, B)
    up = up.reshape(B, 1)
    mp = mp.reshape(B, 1)
    w1t = W1.T
    out = _tc_mlp(gu, gm, up, mp, w1t[:D], w1t[D:],
                  b1.reshape(1, -1), W2.T, b2.reshape(1, 1))
    return out


# speedup vs baseline: 1.0048x; 1.0048x over previous
import functools

import jax
import jax.numpy as jnp
from jax import lax
from jax.experimental import pallas as pl
from jax.experimental.pallas import tpu as pltpu
from jax.experimental.pallas import tpu_sc as plsc

CHUNK = 128
BLKP = 16384


def _relayout_body(t_ref, eye_ref, out_ref):
    x = t_ref[...]
    y = jnp.concatenate([x[:, :BLKP], x[:, BLKP:]], axis=0)
    out_ref[...] = lax.dot_general(
        y, eye_ref[...], (((0,), (0,)), ((), ())),
        preferred_element_type=jnp.float32)


def _relayout(tt):
    E, N = tt.shape
    W = 2 * BLKP
    grid = (N + W - 1) // W
    return pl.pallas_call(
        _relayout_body,
        grid=(grid,),
        in_specs=[
            pl.BlockSpec((E, W), lambda i: (0, i)),
            pl.BlockSpec((2 * E, 2 * E), lambda i: (0, 0)),
        ],
        out_specs=pl.BlockSpec((BLKP, 2 * E), lambda i: (i, 0)),
        out_shape=jax.ShapeDtypeStruct((grid * BLKP, 2 * E), jnp.float32),
        compiler_params=pltpu.CompilerParams(
            dimension_semantics=("parallel",)),
    )(tt, jnp.eye(2 * E, dtype=jnp.float32))


def _map_idx(idx):
    W = 2 * BLKP
    q = idx // W
    s = idx % W
    p = q * BLKP + jnp.where(s < BLKP, s, s - BLKP)
    half = (s >= BLKP).astype(jnp.int32)
    return p, half


@functools.partial(jax.jit, static_argnums=(4,))
def _sc_gather(ut2, uidx, mt2, midx, B):
    info = plsc.get_sparse_core_info()
    NW = info.num_cores * info.num_subcores
    b_per_w = B // NW
    n_ch = b_per_w // CHUNK
    mesh = plsc.VectorSubcoreMesh(core_axis_name="c", subcore_axis_name="s")

    @functools.partial(
        pl.kernel,
        mesh=mesh,
        out_type=(
            jax.ShapeDtypeStruct((B, 128), jnp.float32),
            jax.ShapeDtypeStruct((B, 128), jnp.float32),
        ),
        scratch_types=[
            pltpu.VMEM((n_ch, CHUNK), jnp.int32),
            pltpu.VMEM((n_ch, CHUNK), jnp.int32),
            pltpu.VMEM((2, CHUNK, 128), jnp.float32),
            pltpu.VMEM((2, CHUNK, 128), jnp.float32),
            pltpu.SemaphoreType.DMA,
            pltpu.SemaphoreType.DMA,
        ],
    )
    def k(ut_hbm, uix_hbm, mt_hbm, mix_hbm, gu_hbm, gm_hbm,
          uidx_v, midx_v, ubuf, mbuf, gsem, wsem):
        wid = lax.axis_index("s") * info.num_cores + lax.axis_index("c")
        base = wid * b_per_w
        pltpu.sync_copy(uix_hbm.at[wid], uidx_v)
        pltpu.sync_copy(mix_hbm.at[wid], midx_v)
        writes = [None, None]
        for j in range(n_ch):
            b = j % 2
            if writes[b] is not None:
                for w in writes[b]:
                    w.wait()
            gu = pltpu.async_copy(ut_hbm.at[uidx_v.at[j]], ubuf.at[b], gsem)
            gm = pltpu.async_copy(mt_hbm.at[midx_v.at[j]], mbuf.at[b], gsem)
            gu.wait()
            gm.wait()
            dst = pl.ds(base + j * CHUNK, CHUNK)
            writes[b] = (
                pltpu.async_copy(ubuf.at[b], gu_hbm.at[dst], wsem),
                pltpu.async_copy(mbuf.at[b], gm_hbm.at[dst], wsem),
            )
        for ws in writes:
            if ws is not None:
                for w in ws:
                    w.wait()

    uix3 = uidx.reshape(NW, n_ch, CHUNK)
    mix3 = midx.reshape(NW, n_ch, CHUNK)
    return k(ut2, uix3, mt2, mix3)


def _mlp_body(gu_ref, gm_ref, up_ref, mp_ref, w1u_ref, w1m_ref, b1_ref,
              w2_ref, b2_ref, out_ref):
    ue = jnp.where(up_ref[...] == 1, gu_ref[:, 64:], gu_ref[:, :64])
    me = jnp.where(mp_ref[...] == 1, gm_ref[:, 64:], gm_ref[:, :64])
    h = jnp.dot(ue, w1u_ref[...], preferred_element_type=jnp.float32)
    h = h + jnp.dot(me, w1m_ref[...], preferred_element_type=jnp.float32)
    h = jnp.maximum(h + b1_ref[...], 0.0)
    out_ref[...] = jnp.dot(h, w2_ref[...], preferred_element_type=jnp.float32) + b2_ref[...]


def _tc_mlp(gu, gm, up, mp, w1u, w1m, b1, w2, b2):
    B = gu.shape[0]
    H = w1u.shape[1]
    BLK = 4096
    return pl.pallas_call(
        _mlp_body,
        grid=(B // BLK,),
        in_specs=[
            pl.BlockSpec((BLK, 128), lambda i: (i, 0)),
            pl.BlockSpec((BLK, 128), lambda i: (i, 0)),
            pl.BlockSpec((BLK, 1), lambda i: (i, 0)),
            pl.BlockSpec((BLK, 1), lambda i: (i, 0)),
            pl.BlockSpec((64, H), lambda i: (0, 0)),
            pl.BlockSpec((64, H), lambda i: (0, 0)),
            pl.BlockSpec((1, H), lambda i: (0, 0)),
            pl.BlockSpec((H, 1), lambda i: (0, 0)),
            pl.BlockSpec((1, 1), lambda i: (0, 0)),
        ],
        out_specs=pl.BlockSpec((BLK, 1), lambda i: (i, 0)),
        out_shape=jax.ShapeDtypeStruct((B, 1), jnp.float32),
        compiler_params=pltpu.CompilerParams(
            dimension_semantics=("parallel",)),
    )(gu, gm, up, mp, w1u, w1m, b1, w2, b2)


def kernel(users, movies, user_table, movie_table, W1, b1, W2, b2):
    B = users.shape[0]
    D = user_table.shape[1]
    users = users.astype(jnp.int32)
    movies = movies.astype(jnp.int32)
    rm_u = _relayout(user_table.T)
    rm_m = _relayout(movie_table.T)
    pu, up = _map_idx(users)
    pm, mp = _map_idx(movies)
    gu, gm = _sc_gather(rm_u, pu, rm_m, pm, B)
    up = up.reshape(B, 1)
    mp = mp.reshape(B, 1)
    w1t = W1.T
    out = _tc_mlp(gu, gm, up, mp, w1t[:D], w1t[D:],
                  b1.reshape(1, -1), W2.T, b2.reshape(1, 1))
    return out
